# baseline (speedup 1.0000x reference)
import functools

import jax
import jax.numpy as jnp
from jax import lax
from jax.experimental import pallas as pl
from jax.experimental.pallas import tpu as pltpu
from jax.experimental.pallas import tpu_sc as plsc

_NP = 100352
_BN2 = 2048
_NBLK2 = _NP // _BN2
_BN3 = 512
_NW = 32
_PW = _NP // _NW
_CB = 112


def _min_body(c_ref, o_ref):
    o_ref[...] = jnp.min(c_ref[...], axis=0, keepdims=True)


def _proj_body(shift_ref, lm_ref, intr_ref, xs_ref, ys_ref, zs_ref,
               i0_ref, i1_ref, i2_ref, i3_ref, cnt_ref, *, H, W, HW):
    sx = shift_ref[0, 1]
    sy = shift_ref[0, 2]
    sz = shift_ref[0, 3]
    lx = (xs_ref[0, 0, :] - sx).astype(jnp.float32) + 0.5
    ly = (ys_ref[0, 0, :] - sy).astype(jnp.float32) + 0.5
    lz = (zs_ref[0, 0, :] - sz).astype(jnp.float32) + 0.5
    fx = intr_ref[0, 0]
    fy = intr_ref[0, 1]
    cx = intr_ref[0, 2]
    cy = intr_ref[0, 3]
    cnt = jnp.zeros(lx.shape, jnp.int32)
    outs = (i0_ref, i1_ref, i2_ref, i3_ref)
    for v in range(4):
        cam0 = lx * lm_ref[3 * v + 0, 0] + ly * lm_ref[3 * v + 0, 1] \
            + lz * lm_ref[3 * v + 0, 2] + lm_ref[3 * v + 0, 3]
        cam1 = lx * lm_ref[3 * v + 1, 0] + ly * lm_ref[3 * v + 1, 1] \
            + lz * lm_ref[3 * v + 1, 2] + lm_ref[3 * v + 1, 3]
        z = lx * lm_ref[3 * v + 2, 0] + ly * lm_ref[3 * v + 2, 1] \
            + lz * lm_ref[3 * v + 2, 2] + lm_ref[3 * v + 2, 3]
        zc = jnp.where(jnp.abs(z) < 1e-6, 1e-6, z)
        u = fx * cam0 / zc + cx
        vv = fy * cam1 / zc + cy
        valid = (z >= 0.1 / 0.05) & (z <= 4.0 / 0.05) \
            & (u >= 0.0) & (u < float(W)) & (vv >= 0.0) & (vv < float(H))
        ui = jnp.floor(jnp.clip(u, 0.0, float(W - 1))).astype(jnp.int32)
        vi = jnp.floor(jnp.clip(vv, 0.0, float(H - 1))).astype(jnp.int32)
        lin = v * HW + vi * W + ui
        outs[v][0, 0, :] = jnp.where(valid, lin, 0)
        cnt = cnt + valid.astype(jnp.int32)
    cnt_ref[0, 0, :] = cnt


def _sc_body(feats_hbm, i0_hbm, i1_hbm, i2_hbm, i3_hbm, out_hbm,
             iv0, iv1, iv2, iv3, g0, g1, g2, g3, ov, sem):
    w = lax.axis_index("s") * 2 + lax.axis_index("c")
    base0 = w * _PW

    @pl.loop(0, _PW, step=_CB)
    def _chunk(t):
        base = base0 + t
        pltpu.sync_copy(i0_hbm.at[pl.ds(base, _CB)], iv0)
        pltpu.sync_copy(i1_hbm.at[pl.ds(base, _CB)], iv1)
        pltpu.sync_copy(i2_hbm.at[pl.ds(base, _CB)], iv2)
        pltpu.sync_copy(i3_hbm.at[pl.ds(base, _CB)], iv3)
        c0 = pltpu.async_copy(feats_hbm.at[iv0], g0, sem)
        c1 = pltpu.async_copy(feats_hbm.at[iv1], g1, sem)
        c2 = pltpu.async_copy(feats_hbm.at[iv2], g2, sem)
        c3 = pltpu.async_copy(feats_hbm.at[iv3], g3, sem)
        c0.wait()
        c1.wait()
        c2.wait()
        c3.wait()

        @pl.loop(0, _CB)
        def _row(r):
            for c in range(4):
                sl = pl.ds(c * 16, 16)
                a = g0[r, sl] + g1[r, sl]
                b = g2[r, sl] + g3[r, sl]
                ov[r, sl] = a + b

        pltpu.sync_copy(ov, out_hbm.at[pl.ds(base, _CB)])


def _final_body(sum_ref, cnt_ref, r0_ref, o_ref):
    c = cnt_ref[...].astype(jnp.float32)
    r0 = r0_ref[0:1, :]
    o_ref[...] = (sum_ref[...] - (4.0 - c) * r0) / (c + 1e-4)


def kernel(encoded_2d_features, coords, view_matrix, intrinsic_params):
    B, V, H, W, C = encoded_2d_features.shape
    N = coords.shape[0]
    HW = H * W
    feats_flat = encoded_2d_features.reshape(V * HW, C)

    cr = coords.reshape((N * 4) // 128, 128)
    colmin = pl.pallas_call(
        _min_body,
        out_shape=jax.ShapeDtypeStruct((1, 128), jnp.int32),
    )(cr)
    shift4 = colmin.reshape(32, 4).min(axis=0).reshape(1, 4)

    shift_f = shift4[0, 1:4].astype(jnp.float32)
    lview = view_matrix[0].at[:, :3, 3].add(-shift_f)
    lm = lview[:, :3, :].reshape(12, 4)
    intr = intrinsic_params.reshape(1, 4)
    pad = _NP - N
    xs = jnp.pad(coords[:, 1], (0, pad)).reshape(_NBLK2, 1, _BN2)
    ys = jnp.pad(coords[:, 2], (0, pad)).reshape(_NBLK2, 1, _BN2)
    zs = jnp.pad(coords[:, 3], (0, pad)).reshape(_NBLK2, 1, _BN2)

    blk = pl.BlockSpec((1, 1, _BN2), lambda i: (i, 0, 0))
    smem = pl.BlockSpec(memory_space=pltpu.SMEM)
    i0, i1, i2, i3, cnt = pl.pallas_call(
        functools.partial(_proj_body, H=H, W=W, HW=HW),
        grid=(_NBLK2,),
        in_specs=[smem, smem, smem, blk, blk, blk],
        out_specs=[blk] * 5,
        out_shape=[jax.ShapeDtypeStruct((_NBLK2, 1, _BN2), jnp.int32)] * 5,
    )(shift4, lm, intr, xs, ys, zs)

    mesh = plsc.VectorSubcoreMesh(core_axis_name="c", subcore_axis_name="s")
    sc = pl.kernel(
        _sc_body,
        mesh=mesh,
        compiler_params=pltpu.CompilerParams(use_tc_tiling_on_sc=False),
        out_type=jax.ShapeDtypeStruct((_NP, C), jnp.float32),
        scratch_types=[pltpu.VMEM((_CB,), jnp.int32)] * 4
        + [pltpu.VMEM((_CB, C), jnp.float32)] * 5
        + [pltpu.SemaphoreType.DMA],
    )
    ssum = sc(feats_flat, i0.reshape(_NP), i1.reshape(_NP),
              i2.reshape(_NP), i3.reshape(_NP))

    row0 = jnp.broadcast_to(feats_flat[0:1, :], (8, C))
    out = pl.pallas_call(
        _final_body,
        grid=(_NP // _BN3,),
        in_specs=[
            pl.BlockSpec((_BN3, C), lambda i: (i, 0)),
            pl.BlockSpec((_BN3, 1), lambda i: (i, 0)),
            pl.BlockSpec((8, C), lambda i: (0, 0)),
        ],
        out_specs=pl.BlockSpec((_BN3, C), lambda i: (i, 0)),
        out_shape=jax.ShapeDtypeStruct((N, C), jnp.float32),
    )(ssum, cnt.reshape(_NP, 1), row0)

    return out, cnt.reshape(_NP)[:N]
